# baseline (speedup 1.0000x reference)
import jax
import jax.numpy as jnp
from jax import lax
from jax.experimental import pallas as pl
from jax.experimental.pallas import tpu as pltpu
from jax.experimental.pallas import tpu_sc as plsc

B, L, E = 4096, 50, 300
LP = 56
EG = 384
EP = 304
OUT_W = 3 * EP
NCHUNK = EP // 16
NC, NS = 2, 16
NW = NC * NS
BPW = B // NW
GROUP = 8
HIDDEN, NCLS, NCLS_P = 256, 10, 128


def _pool_body(x0h, x2h, x3h, twh, tbh, tth, outh,
               idx0, idx1, idx2, rows, ostage, sem):
    wid = lax.axis_index("s") * NC + lax.axis_index("c")
    base = wid * BPW
    pltpu.sync_copy(x0h.at[pl.ds(base, BPW)], idx0)
    pltpu.sync_copy(x2h.at[pl.ds(base, BPW)], idx1)
    pltpu.sync_copy(x3h.at[pl.ds(base, BPW)], idx2)
    scale = jnp.float32(1.0 / L)
    zeros16 = jnp.zeros((16,), jnp.float32)

    def group_body(g, carry):
        def row_body(u, carry2):
            r = g * GROUP + u
            for t, (tab, idx) in enumerate(((twh, idx0), (tbh, idx1),
                                            (tth, idx2))):
                pltpu.async_copy(tab.at[idx.at[r]], rows, sem).wait()

                def red(i, acc):
                    return tuple(acc[j] + rows[i, pl.ds(j * 16, 16)]
                                 for j in range(NCHUNK))

                acc = lax.fori_loop(0, L, red, (zeros16,) * NCHUNK)
                for j in range(NCHUNK):
                    ostage[u, pl.ds(t * EP + j * 16, 16)] = acc[j] * scale
            return carry2

        lax.fori_loop(0, GROUP, row_body, 0)
        pltpu.sync_copy(ostage, outh.at[pl.ds(base + g * GROUP, GROUP)])
        return carry

    lax.fori_loop(0, BPW // GROUP, group_body, 0)


def _pooled(x0, x2, x3, emb_word, emb_bi, emb_tri):
    mesh = plsc.VectorSubcoreMesh(core_axis_name="c", subcore_axis_name="s")
    kern = pl.kernel(
        _pool_body,
        mesh=mesh,
        out_type=jax.ShapeDtypeStruct((B, OUT_W), jnp.float32),
        scratch_types=[
            pltpu.VMEM((BPW, LP), jnp.int32),
            pltpu.VMEM((BPW, LP), jnp.int32),
            pltpu.VMEM((BPW, LP), jnp.int32),
            pltpu.VMEM((LP, EG), jnp.float32),
            pltpu.VMEM((GROUP, OUT_W), jnp.float32),
            pltpu.SemaphoreType.DMA,
        ],
    )
    return kern(x0, x2, x3, emb_word, emb_bi, emb_tri)


def _mlp_body(xp, w1, b1, w2, b2, o):
    h = jnp.maximum(
        jnp.dot(xp[...], w1[...], preferred_element_type=jnp.float32) + b1[...], 0.0)
    o[...] = jnp.dot(h, w2[...], preferred_element_type=jnp.float32) + b2[...]


def _mlp(xp, w1p, b1, w2p, b2p):
    bm = 512
    return pl.pallas_call(
        _mlp_body,
        grid=(B // bm,),
        in_specs=[
            pl.BlockSpec((bm, OUT_W), lambda i: (i, 0)),
            pl.BlockSpec((OUT_W, HIDDEN), lambda i: (0, 0)),
            pl.BlockSpec((1, HIDDEN), lambda i: (0, 0)),
            pl.BlockSpec((HIDDEN, NCLS_P), lambda i: (0, 0)),
            pl.BlockSpec((1, NCLS_P), lambda i: (0, 0)),
        ],
        out_specs=pl.BlockSpec((bm, NCLS_P), lambda i: (i, 0)),
        out_shape=jax.ShapeDtypeStruct((B, NCLS_P), jnp.float32),
    )(xp, w1p, b1, w2p, b2p)


def kernel(x0, x2, x3, emb_word, emb_bi, emb_tri, W1, b1, W2, b2):
    pad = ((0, 0), (0, EG - E))
    xpad = ((0, 0), (0, LP - L))
    pooled = _pooled(jnp.pad(x0, xpad), jnp.pad(x2, xpad), jnp.pad(x3, xpad),
                     jnp.pad(emb_word, pad),
                     jnp.pad(emb_bi, pad),
                     jnp.pad(emb_tri, pad))
    w1t = W1.T
    w1p = (jnp.zeros((OUT_W, HIDDEN), jnp.float32)
           .at[0:E].set(w1t[0:E])
           .at[EP:EP + E].set(w1t[E:2 * E])
           .at[2 * EP:2 * EP + E].set(w1t[2 * E:3 * E]))
    w2p = jnp.zeros((HIDDEN, NCLS_P), jnp.float32).at[:, :NCLS].set(W2.T)
    b2p = jnp.zeros((NCLS_P,), jnp.float32).at[:NCLS].set(b2)
    out = _mlp(pooled, w1p, b1.reshape(1, HIDDEN), w2p, b2p.reshape(1, NCLS_P))
    return out[:, :NCLS]


# speedup vs baseline: 1.4514x; 1.4514x over previous
import jax
import jax.numpy as jnp
from jax import lax
from jax.experimental import pallas as pl
from jax.experimental.pallas import tpu as pltpu
from jax.experimental.pallas import tpu_sc as plsc

B, L, E = 4096, 50, 300
LP = 56
EG = 384
EP = 304
OUT_W = 3 * EP
NCHUNK = EP // 16
NC, NS = 2, 16
NW = NC * NS
BPW = B // NW
GROUP = 8
HIDDEN, NCLS, NCLS_P = 256, 10, 128


def _pool_body(x0h, x2h, x3h, twh, tbh, tth, outh,
               idx0, idx1, idx2, rows, ostage, sem):
    wid = lax.axis_index("s") * NC + lax.axis_index("c")
    base = wid * BPW
    pltpu.sync_copy(x0h.at[pl.ds(base, BPW)], idx0)
    pltpu.sync_copy(x2h.at[pl.ds(base, BPW)], idx1)
    pltpu.sync_copy(x3h.at[pl.ds(base, BPW)], idx2)
    scale = jnp.float32(1.0 / L)
    zeros16 = jnp.zeros((16,), jnp.float32)

    def group_body(g, carry):
        def row_body(u, carry2):
            r = g * GROUP + u
            for t, (tab, idx) in enumerate(((twh, idx0), (tbh, idx1),
                                            (tth, idx2))):
                pltpu.async_copy(tab.at[idx.at[r]], rows, sem).wait()

                def red(i, acc):
                    return tuple(acc[j] + rows[i, pl.ds(j * 16, 16)]
                                 for j in range(NCHUNK))

                acc = lax.fori_loop(0, L, red, (zeros16,) * NCHUNK)
                for j in range(NCHUNK):
                    ostage[u, pl.ds(t * EP + j * 16, 16)] = acc[j] * scale
            return carry2

        lax.fori_loop(0, GROUP, row_body, 0)
        pltpu.sync_copy(ostage, outh.at[pl.ds(base + g * GROUP, GROUP)])
        return carry

    lax.fori_loop(0, BPW // GROUP, group_body, 0)


def _pooled(x0, x2, x3, emb_word, emb_bi, emb_tri):
    mesh = plsc.VectorSubcoreMesh(core_axis_name="c", subcore_axis_name="s")
    kern = pl.kernel(
        _pool_body,
        mesh=mesh,
        out_type=jax.ShapeDtypeStruct((B, OUT_W), jnp.float32),
        scratch_types=[
            pltpu.VMEM((BPW, LP), jnp.int32),
            pltpu.VMEM((BPW, LP), jnp.int32),
            pltpu.VMEM((BPW, LP), jnp.int32),
            pltpu.VMEM((LP, EG), jnp.float32),
            pltpu.VMEM((GROUP, OUT_W), jnp.float32),
            pltpu.SemaphoreType.DMA,
        ],
    )
    return kern(x0, x2, x3, emb_word, emb_bi, emb_tri)


def _pad_body(x, o):
    o[...] = jax.lax.pad(x[...], jnp.float32(0.0),
                         ((0, 0, 0), (0, EG - E, 0)))


def _pad_table(tab):
    n = tab.shape[0]
    rb = 512
    grid = (n + rb - 1) // rb
    return pl.pallas_call(
        _pad_body,
        grid=(grid,),
        in_specs=[pl.BlockSpec((rb, E), lambda i: (i, 0))],
        out_specs=pl.BlockSpec((rb, EG), lambda i: (i, 0)),
        out_shape=jax.ShapeDtypeStruct((n, EG), jnp.float32),
    )(tab)


def _mlp_body(xp, w1, b1, w2, b2, o):
    h = jnp.maximum(
        jnp.dot(xp[...], w1[...], preferred_element_type=jnp.float32) + b1[...], 0.0)
    o[...] = jnp.dot(h, w2[...], preferred_element_type=jnp.float32) + b2[...]


def _mlp(xp, w1p, b1, w2p, b2p):
    bm = 512
    return pl.pallas_call(
        _mlp_body,
        grid=(B // bm,),
        in_specs=[
            pl.BlockSpec((bm, OUT_W), lambda i: (i, 0)),
            pl.BlockSpec((OUT_W, HIDDEN), lambda i: (0, 0)),
            pl.BlockSpec((1, HIDDEN), lambda i: (0, 0)),
            pl.BlockSpec((HIDDEN, NCLS_P), lambda i: (0, 0)),
            pl.BlockSpec((1, NCLS_P), lambda i: (0, 0)),
        ],
        out_specs=pl.BlockSpec((bm, NCLS_P), lambda i: (i, 0)),
        out_shape=jax.ShapeDtypeStruct((B, NCLS_P), jnp.float32),
    )(xp, w1p, b1, w2p, b2p)


def kernel(x0, x2, x3, emb_word, emb_bi, emb_tri, W1, b1, W2, b2):
    xpad = ((0, 0), (0, LP - L))
    pooled = _pooled(jnp.pad(x0, xpad), jnp.pad(x2, xpad), jnp.pad(x3, xpad),
                     _pad_table(emb_word),
                     _pad_table(emb_bi),
                     _pad_table(emb_tri))
    w1t = W1.T
    w1p = (jnp.zeros((OUT_W, HIDDEN), jnp.float32)
           .at[0:E].set(w1t[0:E])
           .at[EP:EP + E].set(w1t[E:2 * E])
           .at[2 * EP:2 * EP + E].set(w1t[2 * E:3 * E]))
    w2p = jnp.zeros((HIDDEN, NCLS_P), jnp.float32).at[:, :NCLS].set(W2.T)
    b2p = jnp.zeros((NCLS_P,), jnp.float32).at[:NCLS].set(b2)
    out = _mlp(pooled, w1p, b1.reshape(1, HIDDEN), w2p, b2p.reshape(1, NCLS_P))
    return out[:, :NCLS]
